# baseline (speedup 1.0000x reference)
import functools

import jax
import jax.numpy as jnp
import numpy as np
from jax import lax
from jax.experimental import pallas as pl
from jax.experimental.pallas import tpu as pltpu
from jax.experimental.pallas import tpu_sc as plsc

B = 32
T = 441000
NEW_LEN = 160000
SCALE = np.float32(T / NEW_LEN)

G = 200
IN_CHUNK = 441 * G
OUT_CHUNK = 160 * G
MARGIN = 8
BUF_LEN = IN_CHUNK + 2 * MARGIN
N_CHUNKS = T // IN_CHUNK
VECS = OUT_CHUNK // 16


def _body(wav_hbm, out_hbm, in_v, out_v):
    nc = 2
    row = lax.axis_index("s") * nc + lax.axis_index("c")
    row_in = pl.multiple_of(row * T, 8)
    row_out = pl.multiple_of(row * NEW_LEN, 8)

    for c in range(N_CHUNKS):
        w = c * IN_CHUNK - MARGIN
        if c == 0:
            pltpu.sync_copy(
                wav_hbm.at[pl.ds(row_in, IN_CHUNK + MARGIN)],
                in_v.at[pl.ds(MARGIN, IN_CHUNK + MARGIN)],
            )
        elif c == N_CHUNKS - 1:
            pltpu.sync_copy(
                wav_hbm.at[pl.ds(pl.multiple_of(row_in + w, 8), IN_CHUNK + MARGIN)],
                in_v.at[pl.ds(0, IN_CHUNK + MARGIN)],
            )
        else:
            pltpu.sync_copy(
                wav_hbm.at[pl.ds(pl.multiple_of(row_in + w, 8), BUF_LEN)], in_v
            )

        out_base = c * OUT_CHUNK
        lane = lax.iota(jnp.int32, 16)

        def vec_body(p, _, out_base=out_base, w=w, lane=lane):
            iv = lane + (out_base + p * 16)
            ind = iv.astype(jnp.float32) * SCALE
            lo = ind.astype(jnp.int32)
            frac = ind - lo.astype(jnp.float32)
            bidx = lo - w
            a = plsc.load_gather(in_v, [bidx])
            b = plsc.load_gather(in_v, [bidx + 1])
            out_v[pl.ds(p * 16, 16)] = a * (1.0 - frac) + b * frac
            return 0

        lax.fori_loop(0, VECS, vec_body, 0)
        pltpu.sync_copy(
            out_v, out_hbm.at[pl.ds(pl.multiple_of(row_out + out_base, 8), OUT_CHUNK)]
        )


@functools.cache
def _resample():
    return functools.partial(
        pl.kernel,
        out_type=jax.ShapeDtypeStruct((B * NEW_LEN,), jnp.float32),
        mesh=plsc.VectorSubcoreMesh(core_axis_name="c", subcore_axis_name="s"),
        scratch_types=[
            pltpu.VMEM((BUF_LEN,), jnp.float32),
            pltpu.VMEM((OUT_CHUNK,), jnp.float32),
        ],
        compiler_params=pltpu.CompilerParams(needs_layout_passes=False),
    )(_body)


@jax.jit
def kernel(wav):
    if wav.ndim > 1:
        wav = wav.reshape(wav.shape[0], -1)
    else:
        wav = wav.reshape(1, -1)
    return _resample()(wav.reshape(-1)).reshape(B, NEW_LEN)


# speedup vs baseline: 1.0957x; 1.0957x over previous
import functools

import jax
import jax.numpy as jnp
import numpy as np
from jax import lax
from jax.experimental import pallas as pl
from jax.experimental.pallas import tpu as pltpu
from jax.experimental.pallas import tpu_sc as plsc

B = 32
T = 441000
NEW_LEN = 160000
SCALE = np.float32(T / NEW_LEN)

G = 40
IN_CHUNK = 441 * G
OUT_CHUNK = 160 * G
MARGIN = 8
BUF_LEN = IN_CHUNK + 2 * MARGIN
S_MAX = T - BUF_LEN
N_CHUNKS = T // IN_CHUNK
VECS = OUT_CHUNK // 16
UNROLL = 8


def _body(wav_hbm, out_hbm, in0, in1, ou0, ou1, si0, si1, so0, so1):
    in_bufs = (in0, in1)
    out_bufs = (ou0, ou1)
    in_sems = (si0, si1)
    out_sems = (so0, so1)
    nc = 2
    row = lax.axis_index("s") * nc + lax.axis_index("c")
    row_in = pl.multiple_of(row * T, 8)
    row_out = pl.multiple_of(row * NEW_LEN, 8)
    lane = lax.iota(jnp.int32, 16)

    def window_start(c):
        s = lax.min(lax.max(c * IN_CHUNK - MARGIN, 0), S_MAX)
        return pl.multiple_of(s, 8)

    def issue_in(c, b):
        pltpu.async_copy(
            wav_hbm.at[pl.ds(pl.multiple_of(row_in + window_start(c), 8), BUF_LEN)],
            in_bufs[b],
            in_sems[b],
        )

    def wait_in(b):
        pltpu.make_async_copy(
            wav_hbm.at[pl.ds(row_in, BUF_LEN)], in_bufs[b], in_sems[b]
        ).wait()

    def issue_out(c, b):
        pltpu.async_copy(
            out_bufs[b],
            out_hbm.at[pl.ds(pl.multiple_of(row_out + c * OUT_CHUNK, 8), OUT_CHUNK)],
            out_sems[b],
        )

    def wait_out(b):
        pltpu.make_async_copy(
            out_bufs[b], out_hbm.at[pl.ds(row_out, OUT_CHUNK)], out_sems[b]
        ).wait()

    def compute(c, b):
        s = window_start(c)
        out_base = c * OUT_CHUNK
        in_b = in_bufs[b]
        out_b = out_bufs[b]

        @plsc.parallel_loop(0, VECS, unroll=UNROLL)
        def vec(p):
            iv = lane + (out_base + p * 16)
            ind = iv.astype(jnp.float32) * SCALE
            lo = ind.astype(jnp.int32)
            frac = ind - lo.astype(jnp.float32)
            bidx = lo - s
            a = plsc.load_gather(in_b, [bidx])
            hi = plsc.load_gather(in_b, [bidx + 1])
            out_b[pl.ds(p * 16, 16)] = a * (1.0 - frac) + hi * frac

    issue_in(0, 0)
    issue_in(1, 1)

    @pl.loop(0, N_CHUNKS - 1, step=2)
    def chunk_pair(c0):
        for bb in range(2):
            c = c0 + bb
            wait_in(bb)
            @pl.when(c >= 2)
            def _():
                wait_out(bb)

            compute(c, bb)
            issue_out(c, bb)
            issue_in(c + 2, bb)

    c_last = N_CHUNKS - 1
    wait_in(0)
    wait_out(0)
    compute(c_last, 0)
    issue_out(c_last, 0)
    wait_out(1)
    wait_out(0)
    wait_in(1)


@functools.cache
def _resample():
    return functools.partial(
        pl.kernel,
        out_type=jax.ShapeDtypeStruct((B * NEW_LEN,), jnp.float32),
        mesh=plsc.VectorSubcoreMesh(core_axis_name="c", subcore_axis_name="s"),
        scratch_types=[
            pltpu.VMEM((BUF_LEN,), jnp.float32),
            pltpu.VMEM((BUF_LEN,), jnp.float32),
            pltpu.VMEM((OUT_CHUNK,), jnp.float32),
            pltpu.VMEM((OUT_CHUNK,), jnp.float32),
            pltpu.SemaphoreType.DMA,
            pltpu.SemaphoreType.DMA,
            pltpu.SemaphoreType.DMA,
            pltpu.SemaphoreType.DMA,
        ],
        compiler_params=pltpu.CompilerParams(needs_layout_passes=False),
    )(_body)


@jax.jit
def kernel(wav):
    if wav.ndim > 1:
        wav = wav.reshape(wav.shape[0], -1)
    else:
        wav = wav.reshape(1, -1)
    return _resample()(wav.reshape(-1)).reshape(B, NEW_LEN)
